# baseline (speedup 1.0000x reference)
import functools

import jax
import jax.numpy as jnp
from jax import lax
from jax.experimental import pallas as pl
from jax.experimental.pallas import tpu as pltpu
from jax.experimental.pallas import tpu_sc as plsc

N = 10000
IN_FEATS = 256
H = 512
NUM_CLASSES = 2

NSUB = 16
NPAD = N + 16
E = 160000
EPC = 10240
EPAD = NSUB * EPC
EB = 128
NB = EPC // EB
FC = 128
HB = 40
NCH = H // FC
NDEG = 10240
ZR = NPAD // NSUB
WR = N // NSUB

@functools.cache
def _mesh():
    return plsc.VectorSubcoreMesh(core_axis_name="c", subcore_axis_name="s",
                                  num_cores=2, num_subcores=NSUB)


def _sc_degree(dst_pad):
    @functools.partial(
        pl.kernel,
        out_type=jax.ShapeDtypeStruct((2, NDEG), jnp.float32),
        mesh=_mesh(),
        scratch_types=[
            pltpu.VMEM((EB,), jnp.int32),
            pltpu.VMEM((EB,), jnp.int32),
            pltpu.VMEM((EB,), jnp.float32),
            pltpu.VMEM((1024,), jnp.float32),
            pltpu.VMEM_SHARED((NDEG,), jnp.float32),
            pltpu.SemaphoreType.DMA,
            pltpu.SemaphoreType.DMA,
        ],
    )
    def deg_kernel(dst_h, out_h, idx0_v, idx1_v, ones_v, zeros_v, hist_sh,
                   sem0, sem1):
        cid = lax.axis_index("c")
        sid = lax.axis_index("s")

        @pl.loop(0, EB, step=16)
        def _(i):
            ones_v[pl.ds(i, 16)] = jnp.full((16,), 1.0, jnp.float32)

        @pl.loop(0, 1024, step=16)
        def _(i):
            zeros_v[pl.ds(i, 16)] = jnp.zeros((16,), jnp.float32)

        @pl.when(sid < 10)
        def _():
            pltpu.sync_copy(zeros_v, hist_sh.at[pl.ds(sid * 1024, 1024)])

        plsc.subcore_barrier()

        nb2 = NB // 2
        base = (cid * NSUB + sid) * (nb2 * EB)

        pltpu.async_copy(dst_h.at[pl.ds(base, EB)], idx0_v, sem0)

        @pl.loop(0, nb2, step=2)
        def _(b):
            pltpu.async_copy(dst_h.at[pl.ds(base + (b + 1) * EB, EB)],
                             idx1_v, sem1)
            pltpu.make_async_copy(dst_h.at[pl.ds(base + b * EB, EB)],
                                 idx0_v, sem0).wait()
            pltpu.sync_copy(ones_v, hist_sh.at[idx0_v], add=True)

            @pl.when(b + 2 < nb2)
            def _():
                pltpu.async_copy(dst_h.at[pl.ds(base + (b + 2) * EB, EB)],
                                 idx0_v, sem0)

            pltpu.make_async_copy(dst_h.at[pl.ds(base + (b + 1) * EB, EB)],
                                 idx1_v, sem1).wait()
            pltpu.sync_copy(ones_v, hist_sh.at[idx1_v], add=True)

        plsc.subcore_barrier()

        @pl.when(sid < 10)
        def _():
            pltpu.sync_copy(hist_sh.at[pl.ds(sid * 1024, 1024)],
                            out_h.at[cid, pl.ds(sid * 1024, 1024)])

    return deg_kernel(dst_pad)


def _sc_segsum(fs_chunks, src_pad, dst_pad, zeros_hbm):
    @functools.partial(
        pl.kernel,
        out_type=[jax.ShapeDtypeStruct((N, FC), jnp.float32)
                  for _ in range(NCH)],
        mesh=_mesh(),
        scratch_types=[
            pltpu.VMEM((HB, EB), jnp.int32),
            pltpu.VMEM((NB, EB), jnp.int32),
            [pltpu.VMEM((EB, FC), jnp.float32) for _ in range(2)],
            pltpu.VMEM_SHARED((NPAD, FC), jnp.float32),
            [pltpu.SemaphoreType.DMA for _ in range(2)],
            [pltpu.SemaphoreType.DMA for _ in range(2)],
        ],
    )
    def seg_kernel(fs0, fs1, fs2, fs3, src_h, dst_h, zer_h, o0, o1, o2, o3,
                   si_v, di_v, rows, agg_sh, sem, ssem):
        cid = lax.axis_index("c")
        sid = lax.axis_index("s")

        fss = [fs0, fs1, fs2, fs3]
        outs = [o0, o1, o2, o3]

        pltpu.sync_copy(dst_h.at[sid], di_v)

        for c in range(NCH):
            @pl.when(cid == c // 2)
            def _(c=c):
                fs_h = fss[c]
                out_h = outs[c]
                @pl.when(sid < 15)
                def _():
                    pltpu.sync_copy(zer_h,
                                    agg_sh.at[pl.ds(sid * 632, 632)])

                @pl.when(sid == 15)
                def _():
                    pltpu.sync_copy(zer_h.at[pl.ds(0, 536)],
                                    agg_sh.at[pl.ds(9480, 536)])

                plsc.subcore_barrier()

                def emit(k, h, j):
                    pltpu.make_async_copy(fs_h.at[si_v.at[j]], rows[k],
                                          sem[k]).wait()
                    pltpu.async_copy(rows[k],
                                     agg_sh.at[di_v.at[h * HB + j]],
                                     ssem[k], add=True)

                def refill(k, h, j):
                    pltpu.make_async_copy(
                        rows[k], agg_sh.at[di_v.at[h * HB + j - 2]],
                        ssem[k]).wait()
                    pltpu.async_copy(fs_h.at[si_v.at[j]], rows[k], sem[k])

                @pl.loop(0, NB // HB)
                def _(h):
                    pltpu.sync_copy(src_h.at[sid, pl.ds(h * HB, HB)], si_v)
                    pltpu.async_copy(fs_h.at[si_v.at[0]], rows[0], sem[0])
                    pltpu.async_copy(fs_h.at[si_v.at[1]], rows[1], sem[1])

                    @pl.loop(0, HB - 2, step=2)
                    def _(j):
                        emit(0, h, j)
                        emit(1, h, j + 1)
                        refill(0, h, j + 2)
                        refill(1, h, j + 3)

                    emit(0, h, HB - 2)
                    emit(1, h, HB - 1)
                    pltpu.make_async_copy(
                        rows[0], agg_sh.at[di_v.at[h * HB + HB - 2]],
                        ssem[0]).wait()
                    pltpu.make_async_copy(
                        rows[1], agg_sh.at[di_v.at[h * HB + HB - 1]],
                        ssem[1]).wait()

                plsc.subcore_barrier()
                @pl.when(sid < 15)
                def _():
                    pltpu.sync_copy(agg_sh.at[pl.ds(sid * 632, 632)],
                                    out_h.at[pl.ds(sid * 632, 632)])

                @pl.when(sid == 15)
                def _():
                    pltpu.sync_copy(agg_sh.at[pl.ds(9480, 520)],
                                    out_h.at[pl.ds(9480, 520)])

                plsc.subcore_barrier()

    return seg_kernel(*fs_chunks, src_pad, dst_pad, zeros_hbm)


BN = 2000


def _full(shape):
    return pl.BlockSpec(shape, lambda i: (0,) * len(shape))


def _rows(width):
    return pl.BlockSpec((BN, width), lambda i: (i, 0))


def _tc1_body(x_ref, deg_ref, w1t_ref, b1_ref, w2t_ref, b2_ref,
              h2_ref, dinv_ref, f0_ref, f1_ref, f2_ref, f3_ref):
    dinv = lax.rsqrt(jnp.maximum(deg_ref[...], 1.0))
    h1 = jnp.maximum(
        jnp.dot(x_ref[...], w1t_ref[...],
                preferred_element_type=jnp.float32) + b1_ref[...], 0.0)
    h2 = jnp.maximum(
        jnp.dot(h1, w2t_ref[...],
                preferred_element_type=jnp.float32) + b2_ref[...], 0.0)
    h2_ref[...] = h2
    dinv_ref[...] = dinv
    fs = h2 * dinv
    for k, ref in enumerate((f0_ref, f1_ref, f2_ref, f3_ref)):
        ref[...] = fs[:, k * FC:(k + 1) * FC]


def _tc1(x, deg2d, w1t, b1, w2t, b2):
    return pl.pallas_call(
        _tc1_body,
        grid=(N // BN,),
        in_specs=[_rows(IN_FEATS), _rows(1), _full((IN_FEATS, H)),
                  _full((1, H)), _full((H, H)), _full((1, H))],
        out_specs=[_rows(H), _rows(1)] + [_rows(FC)] * NCH,
        out_shape=[jax.ShapeDtypeStruct((N, H), jnp.float32),
                   jax.ShapeDtypeStruct((N, 1), jnp.float32)]
        + [jax.ShapeDtypeStruct((N, FC), jnp.float32)] * NCH,
    )(x, deg2d, w1t, b1, w2t, b2)


def _tc2_body(h_ref, a0_ref, a1_ref, a2_ref, a3_ref, dinv_ref,
              at_ref, bt_ref, bc_ref,
              c_ref, g0_ref, g1_ref, g2_ref, g3_ref):
    dinv = dinv_ref[...]
    agg = jnp.concatenate(
        [a0_ref[...], a1_ref[...], a2_ref[...], a3_ref[...]], axis=1) * dinv
    c = jnp.maximum(
        jnp.dot(h_ref[...], at_ref[...], preferred_element_type=jnp.float32)
        - jnp.dot(agg, bt_ref[...], preferred_element_type=jnp.float32)
        + bc_ref[...], 0.0)
    c_ref[...] = c
    gs = c * dinv
    for k, ref in enumerate((g0_ref, g1_ref, g2_ref, g3_ref)):
        ref[...] = gs[:, k * FC:(k + 1) * FC]


def _tc2(h, aggs, dinv, at, bt, bc):
    return pl.pallas_call(
        _tc2_body,
        grid=(N // BN,),
        in_specs=[_rows(H)] + [_rows(FC)] * NCH + [_rows(1)]
        + [_full((H, H)), _full((H, H)), _full((1, H))],
        out_specs=[_rows(H)] + [_rows(FC)] * NCH,
        out_shape=[jax.ShapeDtypeStruct((N, H), jnp.float32)]
        + [jax.ShapeDtypeStruct((N, FC), jnp.float32)] * NCH,
    )(h, *aggs, dinv, at, bt, bc)


def _tc3_body(h_ref, a0_ref, a1_ref, a2_ref, a3_ref, dinv_ref,
              at_ref, bt_ref, bc_ref, w3t_ref, b3_ref, w4t_ref, b4_ref,
              out_ref):
    agg = jnp.concatenate(
        [a0_ref[...], a1_ref[...], a2_ref[...], a3_ref[...]],
        axis=1) * dinv_ref[...]
    c2 = jnp.maximum(
        jnp.dot(h_ref[...], at_ref[...], preferred_element_type=jnp.float32)
        - jnp.dot(agg, bt_ref[...], preferred_element_type=jnp.float32)
        + bc_ref[...], 0.0)
    h3 = jnp.maximum(
        jnp.dot(c2, w3t_ref[...], preferred_element_type=jnp.float32)
        + b3_ref[...], 0.0)
    out_ref[...] = (
        jnp.dot(h3, w4t_ref[...], preferred_element_type=jnp.float32)
        + b4_ref[...])


def _tc3(h, aggs, dinv, at, bt, bc, w3t, b3, w4t, b4):
    return pl.pallas_call(
        _tc3_body,
        grid=(N // BN,),
        in_specs=[_rows(H)] + [_rows(FC)] * NCH + [_rows(1)]
        + [_full((H, H)), _full((H, H)), _full((1, H)),
           _full((H, H)), _full((1, H)), _full((H, NUM_CLASSES)),
           _full((1, NUM_CLASSES))],
        out_specs=[_rows(NUM_CLASSES)],
        out_shape=[jax.ShapeDtypeStruct((N, NUM_CLASSES), jnp.float32)],
    )(h, *aggs, dinv, at, bt, bc, w3t, b3, w4t, b4)[0]


def kernel(in_feat, edge_index, W1, b1, W2, b2, Wc1, bc1, Wc2, bc2,
           W3, b3, W4, b4):
    src = edge_index[0]
    dst = edge_index[1]
    npad = EPAD - E
    pad_idx = jnp.arange(npad, dtype=jnp.int32)
    src_p = jnp.concatenate([src, pad_idx % N])
    dst_p = jnp.concatenate([dst, N + pad_idx % 16])

    degs = _sc_degree(dst_p)
    deg2d = (degs[0, :N] + degs[1, :N]).reshape(N, 1)

    h2, dinv, f0, f1, f2, f3 = _tc1(
        in_feat, deg2d, W1.T, b1.reshape(1, H), W2.T, b2.reshape(1, H))

    zeros_hbm = jnp.zeros((632, FC), jnp.float32)
    src3 = src_p.reshape(NSUB, NB, EB)
    dst3 = dst_p.reshape(NSUB, NB, EB)
    a1 = _sc_segsum([f0, f1, f2, f3], src3, dst3, zeros_hbm)
    c1, g0, g1, g2, g3 = _tc2(
        h2, a1, dinv, Wc1[:, :H].T, Wc1[:, H:].T, bc1.reshape(1, H))

    a2 = _sc_segsum([g0, g1, g2, g3], src3, dst3, zeros_hbm)
    out = _tc3(
        c1, a2, dinv, Wc2[:, :H].T, Wc2[:, H:].T, bc2.reshape(1, H),
        W3.T, b3.reshape(1, H), W4.T, b4.reshape(1, NUM_CLASSES))
    return out


# speedup vs baseline: 1.2590x; 1.2590x over previous
import functools

import jax
import jax.numpy as jnp
from jax import lax
from jax.experimental import pallas as pl
from jax.experimental.pallas import tpu as pltpu
from jax.experimental.pallas import tpu_sc as plsc

N = 10000
IN_FEATS = 256
H = 512
NUM_CLASSES = 2

NSUB = 16
NPAD = N + 16
E = 160000
EPC = 10240
EPAD = NSUB * EPC
EB = 128
NB = EPC // EB
FC = 128
HB = 40
NCH = H // FC
NDEG = 10240
ZR = NPAD // NSUB
WR = N // NSUB

@functools.cache
def _mesh():
    return plsc.VectorSubcoreMesh(core_axis_name="c", subcore_axis_name="s",
                                  num_cores=2, num_subcores=NSUB)


def _sc_degree(dst_pad):
    @functools.partial(
        pl.kernel,
        out_type=jax.ShapeDtypeStruct((2, NDEG), jnp.float32),
        mesh=_mesh(),
        scratch_types=[
            pltpu.VMEM((EB,), jnp.int32),
            pltpu.VMEM((EB,), jnp.int32),
            pltpu.VMEM((EB,), jnp.float32),
            pltpu.VMEM((1024,), jnp.float32),
            pltpu.VMEM_SHARED((NDEG,), jnp.float32),
            pltpu.SemaphoreType.DMA,
            pltpu.SemaphoreType.DMA,
        ],
    )
    def deg_kernel(dst_h, out_h, idx0_v, idx1_v, ones_v, zeros_v, hist_sh,
                   sem0, sem1):
        cid = lax.axis_index("c")
        sid = lax.axis_index("s")

        @pl.loop(0, EB, step=16)
        def _(i):
            ones_v[pl.ds(i, 16)] = jnp.full((16,), 1.0, jnp.float32)

        @pl.loop(0, 1024, step=16)
        def _(i):
            zeros_v[pl.ds(i, 16)] = jnp.zeros((16,), jnp.float32)

        @pl.when(sid < 10)
        def _():
            pltpu.sync_copy(zeros_v, hist_sh.at[pl.ds(sid * 1024, 1024)])

        plsc.subcore_barrier()

        nb2 = NB // 2
        base = (cid * NSUB + sid) * (nb2 * EB)

        pltpu.async_copy(dst_h.at[pl.ds(base, EB)], idx0_v, sem0)

        @pl.loop(0, nb2, step=2)
        def _(b):
            pltpu.async_copy(dst_h.at[pl.ds(base + (b + 1) * EB, EB)],
                             idx1_v, sem1)
            pltpu.make_async_copy(dst_h.at[pl.ds(base + b * EB, EB)],
                                 idx0_v, sem0).wait()
            pltpu.sync_copy(ones_v, hist_sh.at[idx0_v], add=True)

            @pl.when(b + 2 < nb2)
            def _():
                pltpu.async_copy(dst_h.at[pl.ds(base + (b + 2) * EB, EB)],
                                 idx0_v, sem0)

            pltpu.make_async_copy(dst_h.at[pl.ds(base + (b + 1) * EB, EB)],
                                 idx1_v, sem1).wait()
            pltpu.sync_copy(ones_v, hist_sh.at[idx1_v], add=True)

        plsc.subcore_barrier()

        @pl.when(sid < 10)
        def _():
            pltpu.sync_copy(hist_sh.at[pl.ds(sid * 1024, 1024)],
                            out_h.at[cid, pl.ds(sid * 1024, 1024)])

    return deg_kernel(dst_pad)


def _sc_segsum(fs_chunks, src_pad, dst_pad, zeros_hbm):
    @functools.partial(
        pl.kernel,
        out_type=[jax.ShapeDtypeStruct((N, FC), jnp.float32)
                  for _ in range(NCH)],
        mesh=_mesh(),
        scratch_types=[
            pltpu.VMEM((HB, EB), jnp.int32),
            pltpu.VMEM((NB, EB), jnp.int32),
            [pltpu.VMEM((EB, FC), jnp.float32) for _ in range(2)],
            pltpu.VMEM_SHARED((NPAD, FC), jnp.float32),
            [pltpu.SemaphoreType.DMA for _ in range(2)],
        ],
    )
    def seg_kernel(fs0, fs1, fs2, fs3, src_h, dst_h, zer_h, o0, o1, o2, o3,
                   si_v, di_v, rows, agg_sh, sem):
        cid = lax.axis_index("c")
        sid = lax.axis_index("s")

        fss = [fs0, fs1, fs2, fs3]
        outs = [o0, o1, o2, o3]

        pltpu.sync_copy(dst_h.at[sid], di_v)

        for c in range(NCH):
            @pl.when(cid == c // 2)
            def _(c=c):
                fs_h = fss[c]
                out_h = outs[c]
                @pl.when(sid < 15)
                def _():
                    pltpu.sync_copy(zer_h,
                                    agg_sh.at[pl.ds(sid * 632, 632)])

                @pl.when(sid == 15)
                def _():
                    pltpu.sync_copy(zer_h.at[pl.ds(0, 536)],
                                    agg_sh.at[pl.ds(9480, 536)])

                plsc.subcore_barrier()

                def start(k, h, j):
                    pltpu.async_copy(fs_h.at[si_v.at[j]], rows[k], sem[k])

                def finish(k, h, j):
                    pltpu.make_async_copy(fs_h.at[si_v.at[j]], rows[k],
                                          sem[k]).wait()
                    pltpu.sync_copy(rows[k], agg_sh.at[di_v.at[h * HB + j]],
                                    add=True)

                @pl.loop(0, NB // HB)
                def _(h):
                    pltpu.sync_copy(src_h.at[sid, pl.ds(h * HB, HB)], si_v)
                    start(0, h, 0)

                    @pl.loop(0, HB - 2, step=2)
                    def _(j):
                        start(1, h, j + 1)
                        finish(0, h, j)
                        start(0, h, j + 2)
                        finish(1, h, j + 1)

                    start(1, h, HB - 1)
                    finish(0, h, HB - 2)
                    finish(1, h, HB - 1)

                plsc.subcore_barrier()
                @pl.when(sid < 15)
                def _():
                    pltpu.sync_copy(agg_sh.at[pl.ds(sid * 632, 632)],
                                    out_h.at[pl.ds(sid * 632, 632)])

                @pl.when(sid == 15)
                def _():
                    pltpu.sync_copy(agg_sh.at[pl.ds(9480, 520)],
                                    out_h.at[pl.ds(9480, 520)])

                plsc.subcore_barrier()

    return seg_kernel(*fs_chunks, src_pad, dst_pad, zeros_hbm)


BN = 2000


def _full(shape):
    return pl.BlockSpec(shape, lambda i: (0,) * len(shape))


def _rows(width):
    return pl.BlockSpec((BN, width), lambda i: (i, 0))


def _tc1_body(x_ref, deg_ref, w1t_ref, b1_ref, w2t_ref, b2_ref,
              h2_ref, dinv_ref, f0_ref, f1_ref, f2_ref, f3_ref):
    dinv = lax.rsqrt(jnp.maximum(deg_ref[...], 1.0))
    h1 = jnp.maximum(
        jnp.dot(x_ref[...], w1t_ref[...],
                preferred_element_type=jnp.float32) + b1_ref[...], 0.0)
    h2 = jnp.maximum(
        jnp.dot(h1, w2t_ref[...],
                preferred_element_type=jnp.float32) + b2_ref[...], 0.0)
    h2_ref[...] = h2
    dinv_ref[...] = dinv
    fs = h2 * dinv
    for k, ref in enumerate((f0_ref, f1_ref, f2_ref, f3_ref)):
        ref[...] = fs[:, k * FC:(k + 1) * FC]


def _tc1(x, deg2d, w1t, b1, w2t, b2):
    return pl.pallas_call(
        _tc1_body,
        grid=(N // BN,),
        in_specs=[_rows(IN_FEATS), _rows(1), _full((IN_FEATS, H)),
                  _full((1, H)), _full((H, H)), _full((1, H))],
        out_specs=[_rows(H), _rows(1)] + [_rows(FC)] * NCH,
        out_shape=[jax.ShapeDtypeStruct((N, H), jnp.float32),
                   jax.ShapeDtypeStruct((N, 1), jnp.float32)]
        + [jax.ShapeDtypeStruct((N, FC), jnp.float32)] * NCH,
    )(x, deg2d, w1t, b1, w2t, b2)


def _tc2_body(h_ref, a0_ref, a1_ref, a2_ref, a3_ref, dinv_ref,
              at_ref, bt_ref, bc_ref,
              c_ref, g0_ref, g1_ref, g2_ref, g3_ref):
    dinv = dinv_ref[...]
    agg = jnp.concatenate(
        [a0_ref[...], a1_ref[...], a2_ref[...], a3_ref[...]], axis=1) * dinv
    c = jnp.maximum(
        jnp.dot(h_ref[...], at_ref[...], preferred_element_type=jnp.float32)
        - jnp.dot(agg, bt_ref[...], preferred_element_type=jnp.float32)
        + bc_ref[...], 0.0)
    c_ref[...] = c
    gs = c * dinv
    for k, ref in enumerate((g0_ref, g1_ref, g2_ref, g3_ref)):
        ref[...] = gs[:, k * FC:(k + 1) * FC]


def _tc2(h, aggs, dinv, at, bt, bc):
    return pl.pallas_call(
        _tc2_body,
        grid=(N // BN,),
        in_specs=[_rows(H)] + [_rows(FC)] * NCH + [_rows(1)]
        + [_full((H, H)), _full((H, H)), _full((1, H))],
        out_specs=[_rows(H)] + [_rows(FC)] * NCH,
        out_shape=[jax.ShapeDtypeStruct((N, H), jnp.float32)]
        + [jax.ShapeDtypeStruct((N, FC), jnp.float32)] * NCH,
    )(h, *aggs, dinv, at, bt, bc)


def _tc3_body(h_ref, a0_ref, a1_ref, a2_ref, a3_ref, dinv_ref,
              at_ref, bt_ref, bc_ref, w3t_ref, b3_ref, w4t_ref, b4_ref,
              out_ref):
    agg = jnp.concatenate(
        [a0_ref[...], a1_ref[...], a2_ref[...], a3_ref[...]],
        axis=1) * dinv_ref[...]
    c2 = jnp.maximum(
        jnp.dot(h_ref[...], at_ref[...], preferred_element_type=jnp.float32)
        - jnp.dot(agg, bt_ref[...], preferred_element_type=jnp.float32)
        + bc_ref[...], 0.0)
    h3 = jnp.maximum(
        jnp.dot(c2, w3t_ref[...], preferred_element_type=jnp.float32)
        + b3_ref[...], 0.0)
    out_ref[...] = (
        jnp.dot(h3, w4t_ref[...], preferred_element_type=jnp.float32)
        + b4_ref[...])


def _tc3(h, aggs, dinv, at, bt, bc, w3t, b3, w4t, b4):
    return pl.pallas_call(
        _tc3_body,
        grid=(N // BN,),
        in_specs=[_rows(H)] + [_rows(FC)] * NCH + [_rows(1)]
        + [_full((H, H)), _full((H, H)), _full((1, H)),
           _full((H, H)), _full((1, H)), _full((H, NUM_CLASSES)),
           _full((1, NUM_CLASSES))],
        out_specs=[_rows(NUM_CLASSES)],
        out_shape=[jax.ShapeDtypeStruct((N, NUM_CLASSES), jnp.float32)],
    )(h, *aggs, dinv, at, bt, bc, w3t, b3, w4t, b4)[0]


def kernel(in_feat, edge_index, W1, b1, W2, b2, Wc1, bc1, Wc2, bc2,
           W3, b3, W4, b4):
    src = edge_index[0]
    dst = edge_index[1]
    npad = EPAD - E
    pad_idx = jnp.arange(npad, dtype=jnp.int32)
    src_p = jnp.concatenate([src, pad_idx % N])
    dst_p = jnp.concatenate([dst, N + pad_idx % 16])

    degs = _sc_degree(dst_p)
    deg2d = (degs[0, :N] + degs[1, :N]).reshape(N, 1)

    h2, dinv, f0, f1, f2, f3 = _tc1(
        in_feat, deg2d, W1.T, b1.reshape(1, H), W2.T, b2.reshape(1, H))

    zeros_hbm = jnp.zeros((632, FC), jnp.float32)
    src3 = src_p.reshape(NSUB, NB, EB)
    dst3 = dst_p.reshape(NSUB, NB, EB)
    a1 = _sc_segsum([f0, f1, f2, f3], src3, dst3, zeros_hbm)
    c1, g0, g1, g2, g3 = _tc2(
        h2, a1, dinv, Wc1[:, :H].T, Wc1[:, H:].T, bc1.reshape(1, H))

    a2 = _sc_segsum([g0, g1, g2, g3], src3, dst3, zeros_hbm)
    out = _tc3(
        c1, a2, dinv, Wc2[:, :H].T, Wc2[:, H:].T, bc2.reshape(1, H),
        W3.T, b3.reshape(1, H), W4.T, b4.reshape(1, NUM_CLASSES))
    return out
